# baseline (device time: 8629 ns/iter reference)
import jax
import jax.numpy as jnp
from jax import lax
from jax.experimental import pallas as pl
from jax.experimental.pallas import tpu as pltpu

_BLK = 64


def kernel(x, dest):
    m_per, n = x.shape
    n_blocks_max = m_per // _BLK
    dest2 = dest.reshape(1, m_per).astype(jnp.int32)

    def body(
        x_hbm, dest_hbm, out_hbm,
        x_ref, dest_ref, theirs_ref, stage_ref, outv_ref,
        load_sems, store_sems, send_sems, recv_sems,
    ):
        my_x = lax.axis_index("x")
        my_y = lax.axis_index("y")
        my_z = lax.axis_index("z")
        peer = (my_x, my_y, 1 - my_z)

        dest_load = pltpu.make_async_copy(dest_hbm, dest_ref, load_sems.at[0])
        x_load = pltpu.make_async_copy(x_hbm, x_ref, load_sems.at[1])
        dest_load.start()
        x_load.start()
        dest_load.wait()

        mine = dest_ref[:, :] == my_z
        mine_f = mine.astype(jnp.float32)
        k_iota = lax.broadcasted_iota(jnp.int32, (m_per, m_per), 0)
        i_iota = lax.broadcasted_iota(jnp.int32, (m_per, m_per), 1)
        tri = jnp.where(k_iota < i_iota, 1.0, 0.0)
        rank_mine = jnp.dot(
            mine_f, tri, preferred_element_type=jnp.float32
        ).astype(jnp.int32)
        idx = lax.broadcasted_iota(jnp.int32, (1, m_per), 1)
        rank_theirs = idx - rank_mine

        m = jnp.sum(mine_f).astype(jnp.int32)
        t = m_per - m

        base_own = jnp.where(my_z == 0, 0, t)
        base_recv = jnp.where(my_z == 0, m, 0)

        barrier_sem = pltpu.get_barrier_semaphore()
        j64 = lax.broadcasted_iota(jnp.int32, (_BLK, m_per), 0)

        x_load.wait()
        rdmas = []
        for c in range(n_blocks_max):
            s = pl.multiple_of(jnp.minimum(c * _BLK, t - _BLK), 16)

            @pl.when(c * _BLK < t)
            def _():
                sel = ((rank_theirs - s == j64) & ~mine).astype(jnp.float32)
                theirs_ref[c * _BLK : (c + 1) * _BLK, :] = jnp.dot(
                    sel, x_ref[:, :], preferred_element_type=jnp.float32
                ).astype(jnp.bfloat16)

            if c == 0:
                pl.semaphore_signal(
                    barrier_sem, inc=1, device_id=peer,
                    device_id_type=pl.DeviceIdType.MESH,
                )
                pl.semaphore_wait(barrier_sem, 1)

            rdma = pltpu.make_async_remote_copy(
                src_ref=theirs_ref.at[pl.ds(c * _BLK, _BLK), :],
                dst_ref=stage_ref.at[pl.ds(s, _BLK), :],
                send_sem=send_sems.at[c],
                recv_sem=recv_sems.at[c],
                device_id=peer,
                device_id_type=pl.DeviceIdType.MESH,
            )
            rdmas.append(rdma)

            @pl.when(c * _BLK < t)
            def _():
                rdma.start()

        own_stores = []
        sel_mine = ((rank_mine + base_own == k_iota) & mine).astype(
            jnp.float32
        )
        outv_ref[:, :] = jnp.dot(
            sel_mine, x_ref[:, :], preferred_element_type=jnp.float32
        )
        for i in range(n_blocks_max):
            s = pl.multiple_of(
                base_own + pl.multiple_of(jnp.minimum(i * _BLK, m - _BLK), 8),
                8,
            )
            st = pltpu.make_async_copy(
                outv_ref.at[pl.ds(s, _BLK), :],
                out_hbm.at[pl.ds(s, _BLK), :],
                store_sems.at[i],
            )
            own_stores.append(st)

            @pl.when(i * _BLK < m)
            def _():
                st.start()

        recv_stores = []
        for i in range(n_blocks_max):
            s = pl.multiple_of(jnp.minimum(i * _BLK, t - _BLK), 16)
            so = pl.multiple_of(base_recv + s, 8)
            recv = pltpu.make_async_remote_copy(
                src_ref=theirs_ref.at[pl.ds(0, _BLK), :],
                dst_ref=stage_ref.at[pl.ds(s, _BLK), :],
                send_sem=send_sems.at[i],
                recv_sem=recv_sems.at[i],
                device_id=peer,
                device_id_type=pl.DeviceIdType.MESH,
            )
            st = pltpu.make_async_copy(
                outv_ref.at[pl.ds(so, _BLK), :],
                out_hbm.at[pl.ds(so, _BLK), :],
                store_sems.at[n_blocks_max + i],
            )
            recv_stores.append(st)

            @pl.when(i * _BLK < t)
            def _():
                recv.wait_recv()
                outv_ref[pl.ds(so, _BLK), :] = stage_ref[
                    pl.ds(s, _BLK), :
                ].astype(jnp.float32)
                st.start()

        for i in range(n_blocks_max):
            @pl.when(i * _BLK < m)
            def _():
                own_stores[i].wait()

            @pl.when(i * _BLK < t)
            def _():
                rdmas[i].wait_send()
                recv_stores[i].wait()

    return pl.pallas_call(
        body,
        out_shape=jax.ShapeDtypeStruct((m_per, n), jnp.float32),
        in_specs=[
            pl.BlockSpec(memory_space=pl.ANY),
            pl.BlockSpec(memory_space=pl.ANY),
        ],
        out_specs=pl.BlockSpec(memory_space=pl.ANY),
        scratch_shapes=[
            pltpu.VMEM((m_per, n), jnp.float32),
            pltpu.VMEM((1, m_per), jnp.int32),
            pltpu.VMEM((m_per, n), jnp.bfloat16),
            pltpu.VMEM((m_per, n), jnp.bfloat16),
            pltpu.VMEM((m_per, n), jnp.float32),
            pltpu.SemaphoreType.DMA((2,)),
            pltpu.SemaphoreType.DMA((2 * n_blocks_max,)),
            pltpu.SemaphoreType.DMA((n_blocks_max,)),
            pltpu.SemaphoreType.DMA((n_blocks_max,)),
        ],
        compiler_params=pltpu.CompilerParams(collective_id=0),
    )(x, dest2)
